# Initial kernel scaffold; baseline (speedup 1.0000x reference)
import jax
import jax.numpy as jnp
from jax.experimental import pallas as pl


def kernel(x, edge_index, edge_type, W1, root1, b1, g1, be1, W2, root2, b2, g2, be2, cls_w, cls_b):
    raise NotImplementedError("write your pallas kernel here")


# speedup vs baseline: 19.1260x; 19.1260x over previous
import functools

import jax
import jax.numpy as jnp
from jax import lax
from jax.experimental import pallas as pl
from jax.experimental.pallas import tpu as pltpu
from jax.experimental.pallas import tpu_sc as plsc

_N = 10000
_E = 320000
_DIN = 128
_DH = 64
_R = 10
_EPS = 1e-5

_NC = 2
_NS = 16
_NW = _NC * _NS
_L = 16

_K = 80
_EPT = _E // _NW
_NCHUNK = _EPT // _K
_EPT_CNT = _E // _NS
_NCHUNK_CNT = _EPT_CNT // _K
_CNTP = 100352
_CPT = _CNTP // _NS
_NP = 10240
_RPT = _NP // _NS
_ZR = 128

_mesh = plsc.VectorSubcoreMesh(core_axis_name="c", subcore_axis_name="s")


def _sc_prep_body(src_hbm, dstn_hbm, et_hbm, gidx_hbm, escale_hbm,
                  src_v, dst_v, typ_v, cidx_v, gidx_v, esc_v, ones_v, cw_v,
                  cnt_sh):
    sid = lax.axis_index("s")
    cid = lax.axis_index("c")
    wid = cid * _NS + sid

    def _zfill(i, carry):
        cw_v[pl.ds(i * _L, _L)] = jnp.zeros((_L,), jnp.float32)
        return carry
    lax.fori_loop(0, _CPT // _L, _zfill, 0)
    for j in range(_K // _L):
        ones_v[pl.ds(j * _L, _L)] = jnp.ones((_L,), jnp.float32)
    pltpu.sync_copy(cw_v, cnt_sh.at[pl.ds(sid * _CPT, _CPT)])
    plsc.subcore_barrier()

    def _cnt_step(c, carry):
        base = sid * _EPT_CNT + c * _K
        pltpu.sync_copy(dstn_hbm.at[pl.ds(base, _K)], dst_v)
        pltpu.sync_copy(et_hbm.at[pl.ds(base, _K)], typ_v)
        for j in range(_K // _L):
            sl = pl.ds(j * _L, _L)
            cidx_v[sl] = typ_v[sl] * _N + dst_v[sl]
        pltpu.sync_copy(ones_v, cnt_sh.at[cidx_v], add=True)
        return carry
    lax.fori_loop(0, _NCHUNK_CNT, _cnt_step, 0)
    plsc.subcore_barrier()

    pltpu.sync_copy(cnt_sh.at[pl.ds(sid * _CPT, _CPT)], cw_v)
    def _inv_step(i, carry):
        sl = pl.ds(i * _L, _L)
        cw_v[sl] = 1.0 / jnp.maximum(cw_v[sl], 1.0)
        return carry
    lax.fori_loop(0, _CPT // _L, _inv_step, 0)
    pltpu.sync_copy(cw_v, cnt_sh.at[pl.ds(sid * _CPT, _CPT)])
    plsc.subcore_barrier()

    def _edge_step(c, carry):
        base = wid * _EPT + c * _K
        pltpu.sync_copy(src_hbm.at[pl.ds(base, _K)], src_v)
        pltpu.sync_copy(dstn_hbm.at[pl.ds(base, _K)], dst_v)
        pltpu.sync_copy(et_hbm.at[pl.ds(base, _K)], typ_v)
        for j in range(_K // _L):
            sl = pl.ds(j * _L, _L)
            t = typ_v[sl]
            gidx_v[sl] = t * _N + src_v[sl]
            cidx_v[sl] = t * _N + dst_v[sl]
        pltpu.sync_copy(gidx_v, gidx_hbm.at[pl.ds(base, _K)])
        pltpu.sync_copy(cnt_sh.at[cidx_v], esc_v)
        pltpu.sync_copy(esc_v, escale_hbm.at[pl.ds(base, _K)])
        return carry
    lax.fori_loop(0, _NCHUNK, _edge_step, 0)


_sc_prep = pl.kernel(
    _sc_prep_body,
    out_type=[jax.ShapeDtypeStruct((_E,), jnp.int32),
              jax.ShapeDtypeStruct((_E,), jnp.float32)],
    mesh=_mesh,
    compiler_params=pltpu.CompilerParams(use_tc_tiling_on_sc=False),
    scratch_types=[
        pltpu.VMEM((_K,), jnp.int32),
        pltpu.VMEM((_K,), jnp.int32),
        pltpu.VMEM((_K,), jnp.int32),
        pltpu.VMEM((_K,), jnp.int32),
        pltpu.VMEM((_K,), jnp.int32),
        pltpu.VMEM((_K,), jnp.float32),
        pltpu.VMEM((_K,), jnp.float32),
        pltpu.VMEM((_CPT,), jnp.float32),
        pltpu.VMEM_SHARED((_CNTP,), jnp.float32),
    ],
)


def _sc_layer_body(xw_hbm, gidx_hbm, escale_hbm, dstn_hbm, msgs_hbm,
                   idx_v, dst_v, esc_v, rows_v, zrows_v, acc_sh):
    sid = lax.axis_index("s")
    cid = lax.axis_index("c")
    wid = cid * _NS + sid

    def _zfill(i, carry):
        for j in range(_DH // _L):
            zrows_v[i, pl.ds(j * _L, _L)] = jnp.zeros((_L,), jnp.float32)
        return carry
    lax.fori_loop(0, _ZR, _zfill, 0)
    for z in range(_RPT // _ZR):
        pltpu.sync_copy(zrows_v, acc_sh.at[pl.ds(sid * _RPT + z * _ZR, _ZR)])
    plsc.subcore_barrier()

    def _edge_step(c, carry):
        base = wid * _EPT + c * _K
        pltpu.sync_copy(gidx_hbm.at[pl.ds(base, _K)], idx_v)
        pltpu.sync_copy(escale_hbm.at[pl.ds(base, _K)], esc_v)
        pltpu.sync_copy(dstn_hbm.at[pl.ds(base, _K)], dst_v)
        pltpu.sync_copy(xw_hbm.at[idx_v], rows_v)

        def _scale_eb(eb, carry2):
            sv = esc_v[pl.ds(eb * _L, _L)]
            for e_lo in range(_L):
                s = jnp.full((_L,), sv[e_lo])
                e = eb * _L + e_lo
                for j in range(_DH // _L):
                    sl = pl.ds(j * _L, _L)
                    rows_v[e, sl] = rows_v[e, sl] * s
            return carry2
        lax.fori_loop(0, _K // _L, _scale_eb, 0)

        pltpu.sync_copy(rows_v, acc_sh.at[dst_v], add=True)
        return carry
    lax.fori_loop(0, _NCHUNK, _edge_step, 0)
    plsc.subcore_barrier()

    pltpu.sync_copy(acc_sh.at[pl.ds(sid * _RPT, _RPT)],
                    msgs_hbm.at[cid, pl.ds(sid * _RPT, _RPT), :])


_sc_layer = pl.kernel(
    _sc_layer_body,
    out_type=jax.ShapeDtypeStruct((_NC, _NP, _DH), jnp.float32),
    mesh=_mesh,
    compiler_params=pltpu.CompilerParams(use_tc_tiling_on_sc=False),
    scratch_types=[
        pltpu.VMEM((_K,), jnp.int32),
        pltpu.VMEM((_K,), jnp.int32),
        pltpu.VMEM((_K,), jnp.float32),
        pltpu.VMEM((_K, _DH), jnp.float32),
        pltpu.VMEM((_ZR, _DH), jnp.float32),
        pltpu.VMEM_SHARED((_NP, _DH), jnp.float32),
    ],
)


def _tc_pre_body(x_ref, w_ref, root_ref, b_ref, xw_ref, hroot_ref):
    r = pl.program_id(0)
    xw_ref[0] = jnp.dot(x_ref[...], w_ref[0],
                        preferred_element_type=jnp.float32)

    @pl.when(r == 0)
    def _():
        hroot_ref[...] = jnp.dot(x_ref[...], root_ref[...],
                                 preferred_element_type=jnp.float32) + b_ref[...]


def _tc_pre(x, w, root, b, d_in):
    return pl.pallas_call(
        _tc_pre_body,
        grid=(_R,),
        in_specs=[
            pl.BlockSpec((_N, d_in), lambda r: (0, 0)),
            pl.BlockSpec((1, d_in, _DH), lambda r: (r, 0, 0)),
            pl.BlockSpec((d_in, _DH), lambda r: (0, 0)),
            pl.BlockSpec((1, _DH), lambda r: (0, 0)),
        ],
        out_specs=[
            pl.BlockSpec((1, _N, _DH), lambda r: (r, 0, 0)),
            pl.BlockSpec((_N, _DH), lambda r: (0, 0)),
        ],
        out_shape=[
            jax.ShapeDtypeStruct((_R, _N, _DH), jnp.float32),
            jax.ShapeDtypeStruct((_N, _DH), jnp.float32),
        ],
    )(x, w, root, b)


def _bn_relu(h, g, be):
    m = jnp.mean(h, axis=0, keepdims=True)
    v = jnp.mean((h - m) * (h - m), axis=0, keepdims=True)
    return jnp.maximum(g * (h - m) * lax.rsqrt(v + _EPS) + be, 0.0)


def _tc_mid_body(hroot_ref, msgs_ref, g_ref, be_ref, w_ref, root_ref, b_ref,
                 xw_ref, hroot2_ref, h_s):
    r = pl.program_id(0)

    @pl.when(r == 0)
    def _():
        h = hroot_ref[...] + msgs_ref[0, :_N] + msgs_ref[1, :_N]
        h = _bn_relu(h, g_ref[...], be_ref[...])
        h_s[...] = h
        hroot2_ref[...] = jnp.dot(h, root_ref[...],
                                  preferred_element_type=jnp.float32) + b_ref[...]

    xw_ref[0] = jnp.dot(h_s[...], w_ref[0], preferred_element_type=jnp.float32)


def _tc_mid(hroot, msgs, g, be, w, root, b):
    return pl.pallas_call(
        _tc_mid_body,
        grid=(_R,),
        in_specs=[
            pl.BlockSpec((_N, _DH), lambda r: (0, 0)),
            pl.BlockSpec((_NC, _NP, _DH), lambda r: (0, 0, 0)),
            pl.BlockSpec((1, _DH), lambda r: (0, 0)),
            pl.BlockSpec((1, _DH), lambda r: (0, 0)),
            pl.BlockSpec((1, _DH, _DH), lambda r: (r, 0, 0)),
            pl.BlockSpec((_DH, _DH), lambda r: (0, 0)),
            pl.BlockSpec((1, _DH), lambda r: (0, 0)),
        ],
        out_specs=[
            pl.BlockSpec((1, _N, _DH), lambda r: (r, 0, 0)),
            pl.BlockSpec((_N, _DH), lambda r: (0, 0)),
        ],
        out_shape=[
            jax.ShapeDtypeStruct((_R, _N, _DH), jnp.float32),
            jax.ShapeDtypeStruct((_N, _DH), jnp.float32),
        ],
        scratch_shapes=[pltpu.VMEM((_N, _DH), jnp.float32)],
    )(hroot, msgs, g, be, w, root, b)


def _tc_final_body(hroot_ref, msgs_ref, g_ref, be_ref, cw_ref, cb_ref, out_ref):
    h = hroot_ref[...] + msgs_ref[0, :_N] + msgs_ref[1, :_N]
    h = _bn_relu(h, g_ref[...], be_ref[...])
    out_ref[...] = jnp.dot(h, cw_ref[...],
                           preferred_element_type=jnp.float32) + cb_ref[...]


def _tc_final(hroot, msgs, g, be, cw, cb):
    return pl.pallas_call(
        _tc_final_body,
        out_shape=jax.ShapeDtypeStruct((_N, 128), jnp.float32),
    )(hroot, msgs, g, be, cw, cb)


def kernel(x, edge_index, edge_type, W1, root1, b1, g1, be1,
           W2, root2, b2, g2, be2, cls_w, cls_b):
    src = edge_index[0]
    dstn = edge_index[1]
    gidx, escale = _sc_prep(src, dstn, edge_type)

    xw1, hroot1 = _tc_pre(x, W1, root1, b1.reshape(1, _DH), _DIN)
    msgs1 = _sc_layer(xw1.reshape(_R * _N, _DH), gidx, escale, dstn)

    xw2, hroot2 = _tc_mid(hroot1, msgs1, g1.reshape(1, _DH),
                          be1.reshape(1, _DH), W2, root2, b2.reshape(1, _DH))
    msgs2 = _sc_layer(xw2.reshape(_R * _N, _DH), gidx, escale, dstn)

    cw_pad = jnp.zeros((_DH, 128), jnp.float32).at[:, :2].set(cls_w)
    cb_pad = jnp.zeros((1, 128), jnp.float32).at[0, :2].set(cls_b)
    out = _tc_final(hroot2, msgs2, g2.reshape(1, _DH), be2.reshape(1, _DH),
                    cw_pad, cb_pad)
    return out[:, :2]


# speedup vs baseline: 38.7828x; 2.0277x over previous
import jax
import jax.numpy as jnp
from jax import lax
from jax.experimental import pallas as pl
from jax.experimental.pallas import tpu as pltpu
from jax.experimental.pallas import tpu_sc as plsc

_N = 10000
_E = 320000
_DIN = 128
_DH = 64
_R = 10
_EPS = 1e-5

_NC = 2
_NS = 16
_NW = _NC * _NS
_L = 16

_K = 80
_ROWS = 4096
_EPAD = _ROWS * _K - _E
_CR = _ROWS // _NS
_ER = _ROWS // _NW
_CNTP = 100352
_CPT = _CNTP // _NS
_TROWS = (_R + 1) * _N
_NP = 10240
_RPT = _NP // _NS
_ZR = 128
_LAG = 8

_mesh = plsc.VectorSubcoreMesh(core_axis_name="c", subcore_axis_name="s")


def _sc_prep_body(src_hbm, dstn_hbm, et_hbm, gidx_hbm, escale_hbm,
                  typ_all, dst_all, ci_all, src_all, gx_all, esc_all,
                  cw_v, ones_v, ssem, gsem, cnt_sh):
    sid = lax.axis_index("s")
    cid = lax.axis_index("c")
    wid = cid * _NS + sid

    def _zfill(i, carry):
        cw_v[pl.ds(i * _L, _L)] = jnp.zeros((_L,), jnp.float32)
        return carry
    lax.fori_loop(0, _CPT // _L, _zfill, 0)
    for j in range(_K // _L):
        ones_v[pl.ds(j * _L, _L)] = jnp.ones((_L,), jnp.float32)
    pltpu.sync_copy(cw_v, cnt_sh.at[pl.ds(sid * _CPT, _CPT)])

    pltpu.sync_copy(et_hbm.at[pl.ds(sid * _CR, _CR)], typ_all)
    pltpu.sync_copy(dstn_hbm.at[pl.ds(sid * _CR, _CR)], dst_all)

    def _ci_step(i, carry):
        for j in range(_K // _L):
            sl = pl.ds(j * _L, _L)
            ci_all[i, sl] = typ_all[i, sl] * _N + dst_all[i, sl]
        return carry
    lax.fori_loop(0, _CR, _ci_step, 0)
    plsc.subcore_barrier()

    def _swait():
        pltpu.make_async_copy(ones_v, cnt_sh.at[ci_all.at[0]], ssem).wait()

    def _cnt_step(c, carry):
        @pl.when(c >= _LAG)
        def _():
            _swait()
        pltpu.async_copy(ones_v, cnt_sh.at[ci_all.at[c]], ssem, add=True)
        return carry
    lax.fori_loop(0, _CR, _cnt_step, 0)
    for _ in range(_LAG):
        _swait()
    plsc.subcore_barrier()

    pltpu.sync_copy(cnt_sh.at[pl.ds(sid * _CPT, _CPT)], cw_v)
    def _inv_step(i, carry):
        sl = pl.ds(i * _L, _L)
        cw_v[sl] = 1.0 / jnp.maximum(cw_v[sl], 1.0)
        return carry
    lax.fori_loop(0, _CPT // _L, _inv_step, 0)
    pltpu.sync_copy(cw_v, cnt_sh.at[pl.ds(sid * _CPT, _CPT)])
    plsc.subcore_barrier()

    pltpu.sync_copy(src_hbm.at[pl.ds(wid * _ER, _ER)], src_all)
    pltpu.sync_copy(et_hbm.at[pl.ds(wid * _ER, _ER)], typ_all.at[pl.ds(0, _ER)])
    pltpu.sync_copy(dstn_hbm.at[pl.ds(wid * _ER, _ER)], dst_all.at[pl.ds(0, _ER)])

    def _gx_step(i, carry):
        for j in range(_K // _L):
            sl = pl.ds(j * _L, _L)
            t = typ_all[i, sl] * _N
            gx_all[i, sl] = t + src_all[i, sl]
            ci_all[i, sl] = t + dst_all[i, sl]
        return carry
    lax.fori_loop(0, _ER, _gx_step, 0)
    pltpu.sync_copy(gx_all, gidx_hbm.at[pl.ds(wid * _ER, _ER)])

    def _gwait():
        pltpu.make_async_copy(cnt_sh.at[ci_all.at[0]], esc_all.at[0], gsem).wait()

    def _esc_step(c, carry):
        @pl.when(c >= _LAG)
        def _():
            _gwait()
        pltpu.async_copy(cnt_sh.at[ci_all.at[c]], esc_all.at[c], gsem)
        return carry
    lax.fori_loop(0, _ER, _esc_step, 0)
    for _ in range(_LAG):
        _gwait()
    pltpu.sync_copy(esc_all, escale_hbm.at[pl.ds(wid * _ER, _ER)])


_sc_prep = pl.kernel(
    _sc_prep_body,
    out_type=[jax.ShapeDtypeStruct((_ROWS, _K), jnp.int32),
              jax.ShapeDtypeStruct((_ROWS, _K), jnp.float32)],
    mesh=_mesh,
    compiler_params=pltpu.CompilerParams(use_tc_tiling_on_sc=False),
    scratch_types=[
        pltpu.VMEM((_CR, _K), jnp.int32),
        pltpu.VMEM((_CR, _K), jnp.int32),
        pltpu.VMEM((_CR, _K), jnp.int32),
        pltpu.VMEM((_ER, _K), jnp.int32),
        pltpu.VMEM((_ER, _K), jnp.int32),
        pltpu.VMEM((_ER, _K), jnp.float32),
        pltpu.VMEM((_CPT,), jnp.float32),
        pltpu.VMEM((_K,), jnp.float32),
        pltpu.SemaphoreType.DMA,
        pltpu.SemaphoreType.DMA,
        pltpu.VMEM_SHARED((_CNTP,), jnp.float32),
    ],
)


def _sc_layer_body(xw_hbm, gidx_hbm, esc_hbm, dstn_hbm, msgs_hbm,
                   g_all, d_all, e_all, rows_v, zrows_v, gsem, ssem, acc_sh):
    sid = lax.axis_index("s")
    cid = lax.axis_index("c")
    wid = cid * _NS + sid
    rbase = wid * _ER

    pltpu.sync_copy(gidx_hbm.at[pl.ds(rbase, _ER)], g_all)
    pltpu.sync_copy(dstn_hbm.at[pl.ds(rbase, _ER)], d_all)
    pltpu.sync_copy(esc_hbm.at[pl.ds(rbase, _ER)], e_all)

    def _zfill(i, carry):
        for j in range(_DH // _L):
            zrows_v[i, pl.ds(j * _L, _L)] = jnp.zeros((_L,), jnp.float32)
        return carry
    lax.fori_loop(0, _ZR, _zfill, 0)
    for z in range(_RPT // _ZR):
        pltpu.sync_copy(zrows_v, acc_sh.at[pl.ds(sid * _RPT + z * _ZR, _ZR)])
    plsc.subcore_barrier()

    def _g_issue(c, b):
        pltpu.async_copy(xw_hbm.at[g_all.at[c]], rows_v.at[b], gsem)

    def _g_wait():
        pltpu.make_async_copy(xw_hbm.at[g_all.at[0]], rows_v.at[0], gsem).wait()

    def _s_wait():
        pltpu.make_async_copy(rows_v.at[0], acc_sh.at[d_all.at[0]], ssem).wait()

    def _scale(c, b):
        def _eb(eb, carry):
            sv = e_all[c, pl.ds(eb * _L, _L)]
            for e_lo in range(_L):
                s = jnp.full((_L,), sv[e_lo])
                e = eb * _L + e_lo
                for j in range(_DH // _L):
                    sl = pl.ds(j * _L, _L)
                    rows_v[b, e, sl] = rows_v[b, e, sl] * s
            return carry
        lax.fori_loop(0, _K // _L, _eb, 0)

    _g_issue(0, 0)
    _g_issue(1, 1)

    def _outer(c4, carry):
        for b in range(4):
            c = c4 * 4 + b
            @pl.when(c >= 2)
            def _():
                _s_wait()
            @pl.when(c <= _ER - 3)
            def _():
                _g_issue(c + 2, (b + 2) % 4)
            _g_wait()
            _scale(c, b)
            pltpu.async_copy(rows_v.at[b], acc_sh.at[d_all.at[c]], ssem,
                             add=True)
        return carry
    lax.fori_loop(0, _ER // 4, _outer, 0)
    _s_wait()
    _s_wait()
    plsc.subcore_barrier()

    pltpu.sync_copy(acc_sh.at[pl.ds(sid * _RPT, _RPT)],
                    msgs_hbm.at[cid, pl.ds(sid * _RPT, _RPT), :])


_sc_layer = pl.kernel(
    _sc_layer_body,
    out_type=jax.ShapeDtypeStruct((_NC, _NP, _DH), jnp.float32),
    mesh=_mesh,
    compiler_params=pltpu.CompilerParams(use_tc_tiling_on_sc=False),
    scratch_types=[
        pltpu.VMEM((_ER, _K), jnp.int32),
        pltpu.VMEM((_ER, _K), jnp.int32),
        pltpu.VMEM((_ER, _K), jnp.float32),
        pltpu.VMEM((4, _K, _DH), jnp.float32),
        pltpu.VMEM((_ZR, _DH), jnp.float32),
        pltpu.SemaphoreType.DMA,
        pltpu.SemaphoreType.DMA,
        pltpu.VMEM_SHARED((_NP, _DH), jnp.float32),
    ],
)


def _tc_pre_body(x_ref, w_ref, root_ref, b_ref, xw_ref, hroot_ref):
    r = pl.program_id(0)

    @pl.when(r < _R)
    def _():
        xw_ref[0] = jnp.dot(x_ref[...], w_ref[0],
                            preferred_element_type=jnp.float32)

    @pl.when(r == _R)
    def _():
        xw_ref[0] = jnp.zeros((_N, _DH), jnp.float32)

    @pl.when(r == 0)
    def _():
        hroot_ref[...] = jnp.dot(x_ref[...], root_ref[...],
                                 preferred_element_type=jnp.float32) + b_ref[...]


def _tc_pre(x, w, root, b, d_in):
    return pl.pallas_call(
        _tc_pre_body,
        grid=(_R + 1,),
        in_specs=[
            pl.BlockSpec((_N, d_in), lambda r: (0, 0)),
            pl.BlockSpec((1, d_in, _DH), lambda r: (jnp.minimum(r, _R - 1), 0, 0)),
            pl.BlockSpec((d_in, _DH), lambda r: (0, 0)),
            pl.BlockSpec((1, _DH), lambda r: (0, 0)),
        ],
        out_specs=[
            pl.BlockSpec((1, _N, _DH), lambda r: (r, 0, 0)),
            pl.BlockSpec((_N, _DH), lambda r: (0, 0)),
        ],
        out_shape=[
            jax.ShapeDtypeStruct((_R + 1, _N, _DH), jnp.float32),
            jax.ShapeDtypeStruct((_N, _DH), jnp.float32),
        ],
    )(x, w, root, b)


def _bn_relu(h, g, be):
    m = jnp.mean(h, axis=0, keepdims=True)
    v = jnp.mean((h - m) * (h - m), axis=0, keepdims=True)
    return jnp.maximum(g * (h - m) * lax.rsqrt(v + _EPS) + be, 0.0)


def _tc_mid_body(hroot_ref, msgs_ref, g_ref, be_ref, w_ref, root_ref, b_ref,
                 xw_ref, hroot2_ref, h_s):
    r = pl.program_id(0)

    @pl.when(r == 0)
    def _():
        h = hroot_ref[...] + msgs_ref[0, :_N] + msgs_ref[1, :_N]
        h = _bn_relu(h, g_ref[...], be_ref[...])
        h_s[...] = h
        hroot2_ref[...] = jnp.dot(h, root_ref[...],
                                  preferred_element_type=jnp.float32) + b_ref[...]

    @pl.when(r < _R)
    def _():
        xw_ref[0] = jnp.dot(h_s[...], w_ref[0],
                            preferred_element_type=jnp.float32)

    @pl.when(r == _R)
    def _():
        xw_ref[0] = jnp.zeros((_N, _DH), jnp.float32)


def _tc_mid(hroot, msgs, g, be, w, root, b):
    return pl.pallas_call(
        _tc_mid_body,
        grid=(_R + 1,),
        in_specs=[
            pl.BlockSpec((_N, _DH), lambda r: (0, 0)),
            pl.BlockSpec((_NC, _NP, _DH), lambda r: (0, 0, 0)),
            pl.BlockSpec((1, _DH), lambda r: (0, 0)),
            pl.BlockSpec((1, _DH), lambda r: (0, 0)),
            pl.BlockSpec((1, _DH, _DH), lambda r: (jnp.minimum(r, _R - 1), 0, 0)),
            pl.BlockSpec((_DH, _DH), lambda r: (0, 0)),
            pl.BlockSpec((1, _DH), lambda r: (0, 0)),
        ],
        out_specs=[
            pl.BlockSpec((1, _N, _DH), lambda r: (r, 0, 0)),
            pl.BlockSpec((_N, _DH), lambda r: (0, 0)),
        ],
        out_shape=[
            jax.ShapeDtypeStruct((_R + 1, _N, _DH), jnp.float32),
            jax.ShapeDtypeStruct((_N, _DH), jnp.float32),
        ],
        scratch_shapes=[pltpu.VMEM((_N, _DH), jnp.float32)],
    )(hroot, msgs, g, be, w, root, b)


def _tc_final_body(hroot_ref, msgs_ref, g_ref, be_ref, cw_ref, cb_ref, out_ref):
    h = hroot_ref[...] + msgs_ref[0, :_N] + msgs_ref[1, :_N]
    h = _bn_relu(h, g_ref[...], be_ref[...])
    out_ref[...] = jnp.dot(h, cw_ref[...],
                           preferred_element_type=jnp.float32) + cb_ref[...]


def _tc_final(hroot, msgs, g, be, cw, cb):
    return pl.pallas_call(
        _tc_final_body,
        out_shape=jax.ShapeDtypeStruct((_N, 128), jnp.float32),
    )(hroot, msgs, g, be, cw, cb)


def kernel(x, edge_index, edge_type, W1, root1, b1, g1, be1,
           W2, root2, b2, g2, be2, cls_w, cls_b):
    ipad = jnp.zeros((_EPAD,), jnp.int32)
    src2 = jnp.concatenate([edge_index[0], ipad]).reshape(_ROWS, _K)
    dst2 = jnp.concatenate([edge_index[1], ipad]).reshape(_ROWS, _K)
    typ2 = jnp.concatenate([edge_type, jnp.full((_EPAD,), _R, jnp.int32)]
                           ).reshape(_ROWS, _K)

    gidx2, esc2 = _sc_prep(src2, dst2, typ2)

    xw1, hroot1 = _tc_pre(x, W1, root1, b1.reshape(1, _DH), _DIN)
    msgs1 = _sc_layer(xw1.reshape(_TROWS, _DH), gidx2, esc2, dst2)

    xw2, hroot2 = _tc_mid(hroot1, msgs1, g1.reshape(1, _DH),
                          be1.reshape(1, _DH), W2, root2, b2.reshape(1, _DH))
    msgs2 = _sc_layer(xw2.reshape(_TROWS, _DH), gidx2, esc2, dst2)

    cw_pad = jnp.zeros((_DH, 128), jnp.float32).at[:, :2].set(cls_w)
    cb_pad = jnp.zeros((1, 128), jnp.float32).at[0, :2].set(cls_b)
    out = _tc_final(hroot2, msgs2, g2.reshape(1, _DH), be2.reshape(1, _DH),
                    cw_pad, cb_pad)
    return out[:, :2]


# speedup vs baseline: 53.0840x; 1.3688x over previous
import jax
import jax.numpy as jnp
from jax import lax
from jax.experimental import pallas as pl
from jax.experimental.pallas import tpu as pltpu
from jax.experimental.pallas import tpu_sc as plsc

_N = 10000
_E = 320000
_DIN = 128
_DH = 64
_R = 10
_EPS = 1e-5

_NC = 2
_NS = 16
_NW = _NC * _NS
_L = 16

_K = 128
_ROWS = 2560
_EPAD = _ROWS * _K - _E
_CR = _ROWS // _NS
_ER = _ROWS // _NW
_CNTP = 100352
_CPT = _CNTP // _NS
_TROWS = (_R + 1) * _N
_NP = 10240
_RPT = _NP // _NS
_ZR = 128
_LAG = 8

_mesh = plsc.VectorSubcoreMesh(core_axis_name="c", subcore_axis_name="s")


def _sc_prep_body(src_hbm, dstn_hbm, et_hbm, gidx_hbm, escale_hbm,
                  typ_all, dst_all, ci_all, src_all, gx_all, esc_all,
                  cw_v, ones_v, ssem, gsem, cnt_sh):
    sid = lax.axis_index("s")
    cid = lax.axis_index("c")
    wid = cid * _NS + sid

    def _zfill(i, carry):
        cw_v[pl.ds(i * _L, _L)] = jnp.zeros((_L,), jnp.float32)
        return carry
    lax.fori_loop(0, _CPT // _L, _zfill, 0)
    for j in range(_K // _L):
        ones_v[pl.ds(j * _L, _L)] = jnp.ones((_L,), jnp.float32)
    pltpu.sync_copy(cw_v, cnt_sh.at[pl.ds(sid * _CPT, _CPT)])

    pltpu.sync_copy(et_hbm.at[pl.ds(sid * _CR, _CR)], typ_all)
    pltpu.sync_copy(dstn_hbm.at[pl.ds(sid * _CR, _CR)], dst_all)

    def _ci_step(i, carry):
        for j in range(_K // _L):
            sl = pl.ds(j * _L, _L)
            ci_all[i, sl] = typ_all[i, sl] * _N + dst_all[i, sl]
        return carry
    lax.fori_loop(0, _CR, _ci_step, 0)
    plsc.subcore_barrier()

    def _swait():
        pltpu.make_async_copy(ones_v, cnt_sh.at[ci_all.at[0]], ssem).wait()

    def _cnt_step(c, carry):
        @pl.when(c >= _LAG)
        def _():
            _swait()
        pltpu.async_copy(ones_v, cnt_sh.at[ci_all.at[c]], ssem, add=True)
        return carry
    lax.fori_loop(0, _CR, _cnt_step, 0)
    for _ in range(_LAG):
        _swait()
    plsc.subcore_barrier()

    pltpu.sync_copy(cnt_sh.at[pl.ds(sid * _CPT, _CPT)], cw_v)
    def _inv_step(i, carry):
        sl = pl.ds(i * _L, _L)
        cw_v[sl] = 1.0 / jnp.maximum(cw_v[sl], 1.0)
        return carry
    lax.fori_loop(0, _CPT // _L, _inv_step, 0)
    pltpu.sync_copy(cw_v, cnt_sh.at[pl.ds(sid * _CPT, _CPT)])
    plsc.subcore_barrier()

    pltpu.sync_copy(src_hbm.at[pl.ds(wid * _ER, _ER)], src_all)
    pltpu.sync_copy(et_hbm.at[pl.ds(wid * _ER, _ER)], typ_all.at[pl.ds(0, _ER)])
    pltpu.sync_copy(dstn_hbm.at[pl.ds(wid * _ER, _ER)], dst_all.at[pl.ds(0, _ER)])

    def _gx_step(i, carry):
        for j in range(_K // _L):
            sl = pl.ds(j * _L, _L)
            t = typ_all[i, sl] * _N
            gx_all[i, sl] = t + src_all[i, sl]
            ci_all[i, sl] = t + dst_all[i, sl]
        return carry
    lax.fori_loop(0, _ER, _gx_step, 0)
    pltpu.sync_copy(gx_all, gidx_hbm.at[pl.ds(wid * _ER, _ER)])

    def _gwait():
        pltpu.make_async_copy(cnt_sh.at[ci_all.at[0]], esc_all.at[0], gsem).wait()

    def _esc_step(c, carry):
        @pl.when(c >= _LAG)
        def _():
            _gwait()
        pltpu.async_copy(cnt_sh.at[ci_all.at[c]], esc_all.at[c], gsem)
        return carry
    lax.fori_loop(0, _ER, _esc_step, 0)
    for _ in range(_LAG):
        _gwait()
    pltpu.sync_copy(esc_all, escale_hbm.at[pl.ds(wid * _ER, _ER)])


_sc_prep = pl.kernel(
    _sc_prep_body,
    out_type=[jax.ShapeDtypeStruct((_ROWS, _K), jnp.int32),
              jax.ShapeDtypeStruct((_ROWS, _K), jnp.float32)],
    mesh=_mesh,
    compiler_params=pltpu.CompilerParams(use_tc_tiling_on_sc=False),
    scratch_types=[
        pltpu.VMEM((_CR, _K), jnp.int32),
        pltpu.VMEM((_CR, _K), jnp.int32),
        pltpu.VMEM((_CR, _K), jnp.int32),
        pltpu.VMEM((_ER, _K), jnp.int32),
        pltpu.VMEM((_ER, _K), jnp.int32),
        pltpu.VMEM((_ER, _K), jnp.float32),
        pltpu.VMEM((_CPT,), jnp.float32),
        pltpu.VMEM((_K,), jnp.float32),
        pltpu.SemaphoreType.DMA,
        pltpu.SemaphoreType.DMA,
        pltpu.VMEM_SHARED((_CNTP,), jnp.float32),
    ],
)


def _sc_layer_body(xw_hbm, gidx_hbm, esc_hbm, dstn_hbm, msgs_hbm,
                   g_all, d_all, e_all, rows_v, srows_v, zrows_v, gsem, ssem,
                   acc_sh):
    sid = lax.axis_index("s")
    cid = lax.axis_index("c")
    wid = cid * _NS + sid
    rbase = wid * _ER

    pltpu.sync_copy(gidx_hbm.at[pl.ds(rbase, _ER)], g_all)
    pltpu.sync_copy(dstn_hbm.at[pl.ds(rbase, _ER)], d_all)
    pltpu.sync_copy(esc_hbm.at[pl.ds(rbase, _ER)], e_all)

    def _zfill(i, carry):
        for j in range(_DH // _L):
            zrows_v[i, pl.ds(j * _L, _L)] = jnp.zeros((_L,), jnp.float32)
        return carry
    lax.fori_loop(0, _ZR, _zfill, 0)
    for z in range(_RPT // _ZR):
        pltpu.sync_copy(zrows_v, acc_sh.at[pl.ds(sid * _RPT + z * _ZR, _ZR)])
    plsc.subcore_barrier()

    def _g_issue(c, b):
        pltpu.async_copy(xw_hbm.at[g_all.at[c]], rows_v.at[b], gsem)

    def _g_wait():
        pltpu.make_async_copy(xw_hbm.at[g_all.at[0]], rows_v.at[0], gsem).wait()

    def _s_wait():
        pltpu.make_async_copy(srows_v.at[0], acc_sh.at[d_all.at[0]], ssem).wait()

    def _scale(c, b):
        def _eb(eb, carry):
            sv = e_all[c, pl.ds(eb * _L, _L)]
            for e_lo in range(_L):
                s = jnp.full((_L,), sv[e_lo])
                e = eb * _L + e_lo
                for g in range(_DH // (2 * _L)):
                    v = rows_v[b, e, pl.ds(g * 2 * _L, 2 * _L)]
                    va, vb = plsc.unpack(v, format=plsc.PackFormat.INTERLEAVED)
                    srows_v[b, e, pl.ds(g * 2 * _L, _L)] = va * s
                    srows_v[b, e, pl.ds(g * 2 * _L + _L, _L)] = vb * s
            return carry
        lax.fori_loop(0, _K // _L, _eb, 0)

    _g_issue(0, 0)
    _g_issue(1, 1)

    def _outer(c4, carry):
        for b in range(4):
            c = c4 * 4 + b
            @pl.when(c >= 2)
            def _():
                _s_wait()
            @pl.when(c <= _ER - 3)
            def _():
                _g_issue(c + 2, (b + 2) % 4)
            _g_wait()
            _scale(c, b)
            pltpu.async_copy(srows_v.at[b], acc_sh.at[d_all.at[c]], ssem,
                             add=True)
        return carry
    lax.fori_loop(0, _ER // 4, _outer, 0)
    _s_wait()
    _s_wait()
    plsc.subcore_barrier()

    pltpu.sync_copy(acc_sh.at[pl.ds(sid * _RPT, _RPT)],
                    msgs_hbm.at[cid, pl.ds(sid * _RPT, _RPT), :])


_sc_layer = pl.kernel(
    _sc_layer_body,
    out_type=jax.ShapeDtypeStruct((_NC, _NP, _DH), jnp.float32),
    mesh=_mesh,
    compiler_params=pltpu.CompilerParams(use_tc_tiling_on_sc=False,
                                         needs_layout_passes=False),
    scratch_types=[
        pltpu.VMEM((_ER, _K), jnp.int32),
        pltpu.VMEM((_ER, _K), jnp.int32),
        pltpu.VMEM((_ER, _K), jnp.float32),
        pltpu.VMEM((4, _K, _DH), jnp.bfloat16),
        pltpu.VMEM((4, _K, _DH), jnp.float32),
        pltpu.VMEM((_ZR, _DH), jnp.float32),
        pltpu.SemaphoreType.DMA,
        pltpu.SemaphoreType.DMA,
        pltpu.VMEM_SHARED((_NP, _DH), jnp.float32),
    ],
)


def _tc_pre_body(x_ref, w_ref, root_ref, b_ref, xw_ref, hroot_ref):
    r = pl.program_id(0)

    @pl.when(r < _R)
    def _():
        xw_ref[0] = jnp.dot(x_ref[...], w_ref[0],
                            preferred_element_type=jnp.float32
                            ).astype(jnp.bfloat16)

    @pl.when(r == _R)
    def _():
        xw_ref[0] = jnp.zeros((_N, _DH), jnp.bfloat16)

    @pl.when(r == 0)
    def _():
        hroot_ref[...] = jnp.dot(x_ref[...], root_ref[...],
                                 preferred_element_type=jnp.float32) + b_ref[...]


def _tc_pre(x, w, root, b, d_in):
    return pl.pallas_call(
        _tc_pre_body,
        grid=(_R + 1,),
        in_specs=[
            pl.BlockSpec((_N, d_in), lambda r: (0, 0)),
            pl.BlockSpec((1, d_in, _DH), lambda r: (jnp.minimum(r, _R - 1), 0, 0)),
            pl.BlockSpec((d_in, _DH), lambda r: (0, 0)),
            pl.BlockSpec((1, _DH), lambda r: (0, 0)),
        ],
        out_specs=[
            pl.BlockSpec((1, _N, _DH), lambda r: (r, 0, 0)),
            pl.BlockSpec((_N, _DH), lambda r: (0, 0)),
        ],
        out_shape=[
            jax.ShapeDtypeStruct((_R + 1, _N, _DH), jnp.bfloat16),
            jax.ShapeDtypeStruct((_N, _DH), jnp.float32),
        ],
    )(x, w, root, b)


def _bn_relu(h, g, be):
    m = jnp.mean(h, axis=0, keepdims=True)
    v = jnp.mean((h - m) * (h - m), axis=0, keepdims=True)
    return jnp.maximum(g * (h - m) * lax.rsqrt(v + _EPS) + be, 0.0)


def _tc_mid_body(hroot_ref, msgs_ref, g_ref, be_ref, w_ref, root_ref, b_ref,
                 xw_ref, hroot2_ref, h_s):
    r = pl.program_id(0)

    @pl.when(r == 0)
    def _():
        h = hroot_ref[...] + msgs_ref[0, :_N] + msgs_ref[1, :_N]
        h = _bn_relu(h, g_ref[...], be_ref[...])
        h_s[...] = h
        hroot2_ref[...] = jnp.dot(h, root_ref[...],
                                  preferred_element_type=jnp.float32) + b_ref[...]

    @pl.when(r < _R)
    def _():
        xw_ref[0] = jnp.dot(h_s[...], w_ref[0],
                            preferred_element_type=jnp.float32
                            ).astype(jnp.bfloat16)

    @pl.when(r == _R)
    def _():
        xw_ref[0] = jnp.zeros((_N, _DH), jnp.bfloat16)


def _tc_mid(hroot, msgs, g, be, w, root, b):
    return pl.pallas_call(
        _tc_mid_body,
        grid=(_R + 1,),
        in_specs=[
            pl.BlockSpec((_N, _DH), lambda r: (0, 0)),
            pl.BlockSpec((_NC, _NP, _DH), lambda r: (0, 0, 0)),
            pl.BlockSpec((1, _DH), lambda r: (0, 0)),
            pl.BlockSpec((1, _DH), lambda r: (0, 0)),
            pl.BlockSpec((1, _DH, _DH), lambda r: (jnp.minimum(r, _R - 1), 0, 0)),
            pl.BlockSpec((_DH, _DH), lambda r: (0, 0)),
            pl.BlockSpec((1, _DH), lambda r: (0, 0)),
        ],
        out_specs=[
            pl.BlockSpec((1, _N, _DH), lambda r: (r, 0, 0)),
            pl.BlockSpec((_N, _DH), lambda r: (0, 0)),
        ],
        out_shape=[
            jax.ShapeDtypeStruct((_R + 1, _N, _DH), jnp.bfloat16),
            jax.ShapeDtypeStruct((_N, _DH), jnp.float32),
        ],
        scratch_shapes=[pltpu.VMEM((_N, _DH), jnp.float32)],
    )(hroot, msgs, g, be, w, root, b)


def _tc_final_body(hroot_ref, msgs_ref, g_ref, be_ref, cw_ref, cb_ref, out_ref):
    h = hroot_ref[...] + msgs_ref[0, :_N] + msgs_ref[1, :_N]
    h = _bn_relu(h, g_ref[...], be_ref[...])
    out_ref[...] = jnp.dot(h, cw_ref[...],
                           preferred_element_type=jnp.float32) + cb_ref[...]


def _tc_final(hroot, msgs, g, be, cw, cb):
    return pl.pallas_call(
        _tc_final_body,
        out_shape=jax.ShapeDtypeStruct((_N, 128), jnp.float32),
    )(hroot, msgs, g, be, cw, cb)


def kernel(x, edge_index, edge_type, W1, root1, b1, g1, be1,
           W2, root2, b2, g2, be2, cls_w, cls_b):
    ipad = jnp.zeros((_EPAD,), jnp.int32)
    src2 = jnp.concatenate([edge_index[0], ipad]).reshape(_ROWS, _K)
    dst2 = jnp.concatenate([edge_index[1], ipad]).reshape(_ROWS, _K)
    typ2 = jnp.concatenate([edge_type, jnp.full((_EPAD,), _R, jnp.int32)]
                           ).reshape(_ROWS, _K)

    gidx2, esc2 = _sc_prep(src2, dst2, typ2)

    cm = jnp.zeros((_DH,), jnp.int32)
    half = jnp.arange(_DH // 4)
    for g in range(_DH // (2 * _L)):
        cm = cm.at[32 * g + 2 * half].set(32 * g + half)
        cm = cm.at[32 * g + 2 * half + 1].set(32 * g + _L + half)
    W1p = W1[:, :, cm]
    W2p = W2[:, :, cm]

    xw1, hroot1 = _tc_pre(x, W1p, root1, b1.reshape(1, _DH), _DIN)
    msgs1 = _sc_layer(xw1.reshape(_TROWS, _DH), gidx2, esc2, dst2)

    xw2, hroot2 = _tc_mid(hroot1, msgs1, g1.reshape(1, _DH),
                          be1.reshape(1, _DH), W2p, root2, b2.reshape(1, _DH))
    msgs2 = _sc_layer(xw2.reshape(_TROWS, _DH), gidx2, esc2, dst2)

    cw_pad = jnp.zeros((_DH, 128), jnp.float32).at[:, :2].set(cls_w)
    cb_pad = jnp.zeros((1, 128), jnp.float32).at[0, :2].set(cls_b)
    out = _tc_final(hroot2, msgs2, g2.reshape(1, _DH), be2.reshape(1, _DH),
                    cw_pad, cb_pad)
    return out[:, :2]
